# baseline (speedup 1.0000x reference)
import functools

import jax
import jax.numpy as jnp
from jax import lax
from jax.experimental import pallas as pl
from jax.experimental.pallas import tpu as pltpu, tpu_sc as plsc

D_VOCAB = 1000000
D_MODEL = 64
BATCH = 16384


def _embed_call(idx2, W_flat):
    info = plsc.get_sparse_core_info()
    nw = info.num_cores * info.num_subcores
    b_per_w = BATCH // nw
    mesh = plsc.VectorSubcoreMesh(core_axis_name="c", subcore_axis_name="s")

    @functools.partial(
        pl.kernel,
        mesh=mesh,
        out_type=jax.ShapeDtypeStruct((D_MODEL, BATCH), jnp.float32),
        scratch_types=[
            pltpu.VMEM((D_MODEL, b_per_w), jnp.int32),
            pltpu.VMEM((D_MODEL, b_per_w), jnp.float32),
            pltpu.SemaphoreType.DMA,
        ],
        compiler_params=pltpu.CompilerParams(
            use_tc_tiling_on_sc=False, needs_layout_passes=False
        ),
    )
    def k(idx_hbm, table_hbm, out_hbm, idx_v, rows_v, sem):
        wid = lax.axis_index("s") * info.num_cores + lax.axis_index("c")
        base = wid * b_per_w
        pltpu.sync_copy(idx_hbm.at[:, pl.ds(base, b_per_w)], idx_v)
        copies = [
            pltpu.async_copy(table_hbm.at[idx_v.at[d]], rows_v.at[d], sem)
            for d in range(D_MODEL)
        ]
        for c in copies:
            c.wait()
        pltpu.sync_copy(rows_v, out_hbm.at[:, pl.ds(base, b_per_w)])

    return k(idx2, W_flat)


def kernel(tokens, W_E):
    idx2 = tokens[None, :].astype(jnp.int32) + (
        jnp.arange(D_MODEL, dtype=jnp.int32) * D_VOCAB
    )[:, None]
    W_flat = W_E.T.reshape(-1)
    out_t = _embed_call(idx2, W_flat)
    return out_t.T


# speedup vs baseline: 3.5884x; 3.5884x over previous
import functools

import jax
import jax.numpy as jnp
from jax import lax
from jax.experimental import pallas as pl
from jax.experimental.pallas import tpu as pltpu, tpu_sc as plsc

D_MODEL = 64
BATCH = 16384
HALF = D_MODEL // 2


def _embed_call(tokens_i32, W_a, W_b):
    info = plsc.get_sparse_core_info()
    nw = info.num_cores * info.num_subcores
    b_per_w = BATCH // nw
    mesh = plsc.VectorSubcoreMesh(core_axis_name="c", subcore_axis_name="s")

    @functools.partial(
        pl.kernel,
        mesh=mesh,
        out_type=jax.ShapeDtypeStruct((BATCH, D_MODEL), jnp.float32),
        scratch_types=[
            pltpu.VMEM((b_per_w,), jnp.int32),
            pltpu.VMEM((b_per_w, HALF), jnp.float32),
            pltpu.VMEM((b_per_w, HALF), jnp.float32),
            pltpu.SemaphoreType.DMA,
        ],
        compiler_params=pltpu.CompilerParams(
            use_tc_tiling_on_sc=False, needs_layout_passes=False
        ),
    )
    def k(idx_hbm, ta_hbm, tb_hbm, out_hbm, idx_v, rows_a, rows_b, sem):
        wid = lax.axis_index("s") * info.num_cores + lax.axis_index("c")
        base = wid * b_per_w
        pltpu.sync_copy(idx_hbm.at[pl.ds(base, b_per_w)], idx_v)
        ca = pltpu.async_copy(ta_hbm.at[idx_v], rows_a, sem)
        cb = pltpu.async_copy(tb_hbm.at[idx_v], rows_b, sem)
        ca.wait()
        cb.wait()
        pltpu.sync_copy(rows_a, out_hbm.at[pl.ds(base, b_per_w), pl.ds(0, HALF)])
        pltpu.sync_copy(rows_b, out_hbm.at[pl.ds(base, b_per_w), pl.ds(HALF, HALF)])

    return k(tokens_i32, W_a, W_b)


def kernel(tokens, W_E):
    return _embed_call(
        tokens.astype(jnp.int32), W_E[:, :HALF], W_E[:, HALF:]
    )


# speedup vs baseline: 7.4589x; 2.0786x over previous
import functools

import jax
import jax.numpy as jnp
from jax import lax
from jax.experimental import pallas as pl
from jax.experimental.pallas import tpu as pltpu, tpu_sc as plsc

D_VOCAB = 1000000
D_MODEL = 64
BATCH = 16384
TBLK = 8192


def _transpose_call(W_T):
    def tkern(x_ref, o_ref):
        o_ref[...] = x_ref[...].T

    grid = (D_VOCAB + TBLK - 1) // TBLK
    return pl.pallas_call(
        tkern,
        grid=(grid,),
        in_specs=[pl.BlockSpec((D_MODEL, TBLK), lambda i: (0, i))],
        out_specs=pl.BlockSpec((TBLK, D_MODEL), lambda i: (i, 0)),
        out_shape=jax.ShapeDtypeStruct((D_VOCAB, D_MODEL), jnp.float32),
    )(W_T)


def _embed_call(tokens_i32, W_rows):
    info = plsc.get_sparse_core_info()
    nw = info.num_cores * info.num_subcores
    b_per_w = BATCH // nw
    mesh = plsc.VectorSubcoreMesh(core_axis_name="c", subcore_axis_name="s")

    @functools.partial(
        pl.kernel,
        mesh=mesh,
        out_type=jax.ShapeDtypeStruct((BATCH, D_MODEL), jnp.float32),
        scratch_types=[
            pltpu.VMEM((b_per_w,), jnp.int32),
            pltpu.VMEM((b_per_w, D_MODEL), jnp.float32),
            pltpu.SemaphoreType.DMA,
        ],
        compiler_params=pltpu.CompilerParams(
            use_tc_tiling_on_sc=False, needs_layout_passes=False
        ),
    )
    def k(idx_hbm, table_hbm, out_hbm, idx_v, rows_v, sem):
        wid = lax.axis_index("s") * info.num_cores + lax.axis_index("c")
        base = wid * b_per_w
        pltpu.sync_copy(idx_hbm.at[pl.ds(base, b_per_w)], idx_v)
        pltpu.async_copy(table_hbm.at[idx_v], rows_v, sem).wait()
        pltpu.sync_copy(rows_v, out_hbm.at[pl.ds(base, b_per_w)])

    return k(tokens_i32, W_rows)


def kernel(tokens, W_E):
    W_rows = _transpose_call(W_E.T)
    return _embed_call(tokens.astype(jnp.int32), W_rows)
